# baseline (speedup 1.0000x reference)
import functools

import jax
import jax.numpy as jnp
from jax import lax
from jax.experimental import pallas as pl
from jax.experimental.pallas import tpu as pltpu
from jax.experimental.pallas import tpu_sc as plsc

_NC = 2
_NS = 16
_W = 16
_G = 312
_ZH = _G // 2


def _make_plan(b):
    rps = ((b // _NS) // 8) * 8
    tail = b - _NS * rps
    nch = rps // _G
    rem = rps - nch * _G
    assert tail % 8 == 0 and tail <= 128 and rem <= 128
    assert nch >= 2 and nch % 2 == 0
    return rps, tail, nch, rem


def _sc_body(plan, cand, graph, idx, iota, out, acc, a0, a1, b0, b1, i0, i1,
             d0, d1, idx_rem, idx_tail, idn_rem, idn_tail, zsrc,
             la0, la1, sc0, sc1, zs, wb0, wb1):
    rps, tail, nch, rem = plan
    npass = cand.shape[1] // (_NC * _W)
    c = lax.axis_index("c")
    s = lax.axis_index("s")
    row0 = s * rps
    t0 = _NS * rps
    a_ = (a0, a1)
    b_ = (b0, b1)
    i_ = (i0, i1)
    d_ = (d0, d1)
    la = (la0, la1)
    sc = (sc0, sc1)
    wb = (wb0, wb1)

    if rem:
        pltpu.sync_copy(idx.at[pl.ds(row0 + nch * _G, rem)], idx_rem)
        pltpu.sync_copy(iota.at[pl.ds(row0 + nch * _G, rem)], idn_rem)
    if tail:
        @pl.when(s == _NS - 1)
        def _():
            pltpu.sync_copy(idx.at[pl.ds(t0, tail)], idx_tail)
            pltpu.sync_copy(iota.at[pl.ds(t0, tail)], idn_tail)

    @plsc.parallel_loop(0, _ZH, step=8, unroll=2)
    def _zero(i):
        for k in range(8):
            zsrc[i + k, :] = jnp.zeros((_W,), jnp.float32)

    def _add_rows(dst, src, n):
        @plsc.parallel_loop(0, n, step=8, unroll=2)
        def _body(i):
            for k in range(8):
                dst[i + k, :] = dst[i + k, :] + src[i + k, :]

    def one_pass(p, carry):
        col0 = (c * npass + p) * _W

        @pl.when(p == 0)
        def _():
            def zfill(k, carry2):
                for h in (0, 1):
                    pltpu.async_copy(
                        zsrc, acc.at[pl.ds(row0 + k * _G + h * _ZH, _ZH), :],
                        zs)
                return carry2
            lax.fori_loop(0, nch, zfill, 0)
            if rem:
                pltpu.async_copy(zsrc.at[pl.ds(0, rem), :],
                                 acc.at[pl.ds(row0 + nch * _G, rem), :], zs)
            if tail:
                @pl.when(s == _NS - 1)
                def _():
                    pltpu.async_copy(zsrc.at[pl.ds(0, tail), :],
                                     acc.at[pl.ds(t0, tail), :], zs)

        @pl.when(p > 0)
        def _():
            colq = (c * npass + p - 1) * _W
            for u in (0, 1):
                kq = nch - 2 + u
                pltpu.make_async_copy(
                    acc.at[pl.ds(row0 + kq * _G, _G), :],
                    out.at[pl.ds(row0 + kq * _G, _G), pl.ds(colq, _W)],
                    wb[u]).wait()
                for h in (0, 1):
                    pltpu.async_copy(
                        zsrc,
                        acc.at[pl.ds(row0 + kq * _G + h * _ZH, _ZH), :], zs)

        def zdrain(k, carry2):
            for h in (0, 1):
                pltpu.make_async_copy(
                    zsrc, acc.at[pl.ds(row0 + k * _G + h * _ZH, _ZH), :],
                    zs).wait()
            return carry2
        lax.fori_loop(0, nch, zdrain, 0)
        if rem:
            pltpu.make_async_copy(zsrc.at[pl.ds(0, rem), :],
                                  acc.at[pl.ds(row0 + nch * _G, rem), :],
                                  zs).wait()
        if tail:
            @pl.when(s == _NS - 1)
            def _():
                pltpu.make_async_copy(zsrc.at[pl.ds(0, tail), :],
                                      acc.at[pl.ds(t0, tail), :], zs).wait()

        def issue_loads(j, u):
            base = row0 + j * _G
            pltpu.async_copy(idx.at[pl.ds(base, _G)], i_[u], la[u])
            pltpu.async_copy(iota.at[pl.ds(base, _G)], d_[u], la[u])
            pltpu.async_copy(cand.at[pl.ds(base, _G), pl.ds(col0, _W)],
                             a_[u], la[u])
            pltpu.async_copy(graph.at[pl.ds(base, _G), pl.ds(col0, _W)],
                             b_[u], la[u])

        def wait_loads(j, u):
            base = row0 + j * _G
            pltpu.make_async_copy(idx.at[pl.ds(base, _G)], i_[u], la[u]).wait()
            pltpu.make_async_copy(iota.at[pl.ds(base, _G)], d_[u],
                                  la[u]).wait()
            pltpu.make_async_copy(cand.at[pl.ds(base, _G), pl.ds(col0, _W)],
                                  a_[u], la[u]).wait()
            pltpu.make_async_copy(graph.at[pl.ds(base, _G), pl.ds(col0, _W)],
                                  b_[u], la[u]).wait()

        def issue_scatter(u):
            pltpu.async_copy(b_[u], acc.at[i_[u]], sc[u], add=True)
            pltpu.async_copy(a_[u], acc.at[i_[u]], sc[u], add=True)
            pltpu.async_copy(a_[u], acc.at[d_[u]], sc[u], add=True)

        def wait_scatter(u):
            pltpu.make_async_copy(b_[u], acc.at[i_[u]], sc[u]).wait()
            pltpu.make_async_copy(a_[u], acc.at[i_[u]], sc[u]).wait()
            pltpu.make_async_copy(a_[u], acc.at[d_[u]], sc[u]).wait()

        issue_loads(0, 0)
        plsc.subcore_barrier()
        wait_loads(0, 0)
        issue_loads(1, 1)
        issue_scatter(0)

        def pair(j2, carry2):
            j = 1 + 2 * j2
            wait_loads(j, 1)
            wait_scatter(0)
            issue_loads(j + 1, 0)
            issue_scatter(1)
            wait_loads(j + 1, 0)
            wait_scatter(1)
            issue_loads(j + 2, 1)
            issue_scatter(0)
            return carry2
        lax.fori_loop(0, (nch - 2) // 2, pair, 0)

        wait_loads(nch - 1, 1)
        wait_scatter(0)
        issue_scatter(1)

        if rem:
            roff = row0 + nch * _G
            pltpu.sync_copy(cand.at[pl.ds(roff, rem), pl.ds(col0, _W)],
                            a0.at[pl.ds(0, rem), :])
            pltpu.sync_copy(graph.at[pl.ds(roff, rem), pl.ds(col0, _W)],
                            b0.at[pl.ds(0, rem), :])
            pltpu.sync_copy(b0.at[pl.ds(0, rem), :],
                            acc.at[idx_rem], add=True)
            pltpu.sync_copy(a0.at[pl.ds(0, rem), :],
                            acc.at[idx_rem], add=True)
            pltpu.sync_copy(a0.at[pl.ds(0, rem), :],
                            acc.at[idn_rem], add=True)

        if tail:
            @pl.when(s == _NS - 1)
            def _():
                pltpu.sync_copy(cand.at[pl.ds(t0, tail), pl.ds(col0, _W)],
                                a0.at[pl.ds(0, tail), :])
                pltpu.sync_copy(graph.at[pl.ds(t0, tail), pl.ds(col0, _W)],
                                b0.at[pl.ds(0, tail), :])
                pltpu.sync_copy(b0.at[pl.ds(0, tail), :],
                                acc.at[idx_tail], add=True)
                pltpu.sync_copy(a0.at[pl.ds(0, tail), :],
                                acc.at[idx_tail], add=True)
                pltpu.sync_copy(a0.at[pl.ds(0, tail), :],
                                acc.at[idn_tail], add=True)

        wait_scatter(1)
        plsc.subcore_barrier()

        if rem:
            roff = row0 + nch * _G
            pltpu.sync_copy(acc.at[pl.ds(roff, rem), :],
                            b1.at[pl.ds(0, rem), :])
            pltpu.sync_copy(b1.at[pl.ds(0, rem), :],
                            out.at[pl.ds(roff, rem), pl.ds(col0, _W)])
            pltpu.async_copy(zsrc.at[pl.ds(0, rem), :],
                             acc.at[pl.ds(roff, rem), :], zs)
        if tail:
            @pl.when(s == _NS - 1)
            def _():
                pltpu.sync_copy(acc.at[pl.ds(t0, tail), :],
                                b1.at[pl.ds(0, tail), :])
                pltpu.sync_copy(b1.at[pl.ds(0, tail), :],
                                out.at[pl.ds(t0, tail), pl.ds(col0, _W)])
                pltpu.async_copy(zsrc.at[pl.ds(0, tail), :],
                                 acc.at[pl.ds(t0, tail), :], zs)

        def wb_store(k, u):
            pltpu.async_copy(acc.at[pl.ds(row0 + k * _G, _G), :],
                             out.at[pl.ds(row0 + k * _G, _G),
                                    pl.ds(col0, _W)], wb[u])

        def wb_wait_store(k, u):
            pltpu.make_async_copy(acc.at[pl.ds(row0 + k * _G, _G), :],
                                  out.at[pl.ds(row0 + k * _G, _G),
                                         pl.ds(col0, _W)], wb[u]).wait()

        def zfill_chunk(k):
            for h in (0, 1):
                pltpu.async_copy(
                    zsrc, acc.at[pl.ds(row0 + k * _G + h * _ZH, _ZH), :], zs)

        wb_store(0, 0)
        wb_store(1, 1)

        def wb_pair(k2, carry2):
            k = 2 + 2 * k2
            wb_wait_store(k - 2, 0)
            zfill_chunk(k - 2)
            wb_store(k, 0)
            wb_wait_store(k - 1, 1)
            zfill_chunk(k - 1)
            wb_store(k + 1, 1)
            return carry2
        lax.fori_loop(0, (nch - 2) // 2, wb_pair, 0)
        return carry

    lax.fori_loop(0, npass, one_pass, 0)

    colz = (c * npass + npass - 1) * _W
    for u in (0, 1):
        kq = nch - 2 + u
        pltpu.make_async_copy(acc.at[pl.ds(row0 + kq * _G, _G), :],
                              out.at[pl.ds(row0 + kq * _G, _G),
                                     pl.ds(colz, _W)], wb[u]).wait()

    def zdrain_f(k, carry2):
        for h in (0, 1):
            pltpu.make_async_copy(
                zsrc, acc.at[pl.ds(row0 + k * _G + h * _ZH, _ZH), :],
                zs).wait()
        return carry2
    lax.fori_loop(0, nch - 2, zdrain_f, 0)
    if rem:
        pltpu.make_async_copy(zsrc.at[pl.ds(0, rem), :],
                              acc.at[pl.ds(row0 + nch * _G, rem), :],
                              zs).wait()
    if tail:
        @pl.when(s == _NS - 1)
        def _():
            pltpu.make_async_copy(zsrc.at[pl.ds(0, tail), :],
                                  acc.at[pl.ds(t0, tail), :], zs).wait()


@jax.jit
def _scatter_add(cand, graph, idx_flat):
    b, d = cand.shape
    plan = _make_plan(b)
    rps, tail, nch, rem = plan
    iota = jnp.arange(b, dtype=jnp.int32)
    mesh = plsc.VectorSubcoreMesh(core_axis_name="c", subcore_axis_name="s")
    f = pl.kernel(
        functools.partial(_sc_body, plan),
        out_type=jax.ShapeDtypeStruct((b, d), jnp.float32),
        mesh=mesh,
        compiler_params=pltpu.CompilerParams(use_tc_tiling_on_sc=False),
        scratch_types=[
            pltpu.VMEM_SHARED((b, _W), jnp.float32),
            pltpu.VMEM((_G, _W), jnp.float32),
            pltpu.VMEM((_G, _W), jnp.float32),
            pltpu.VMEM((_G, _W), jnp.float32),
            pltpu.VMEM((_G, _W), jnp.float32),
            pltpu.VMEM((_G,), jnp.int32),
            pltpu.VMEM((_G,), jnp.int32),
            pltpu.VMEM((_G,), jnp.int32),
            pltpu.VMEM((_G,), jnp.int32),
            pltpu.VMEM((max(rem, 1),), jnp.int32),
            pltpu.VMEM((max(tail, 1),), jnp.int32),
            pltpu.VMEM((max(rem, 1),), jnp.int32),
            pltpu.VMEM((max(tail, 1),), jnp.int32),
            pltpu.VMEM((_ZH, _W), jnp.float32),
            pltpu.SemaphoreType.DMA,
            pltpu.SemaphoreType.DMA,
            pltpu.SemaphoreType.DMA,
            pltpu.SemaphoreType.DMA,
            pltpu.SemaphoreType.DMA,
            pltpu.SemaphoreType.DMA,
            pltpu.SemaphoreType.DMA,
        ],
    )
    return f(cand, graph, idx_flat, iota)


def kernel(candidate_rep, graph_rep, graph_sizes, put_indices):
    del graph_sizes
    idx_flat = put_indices.reshape(-1)
    out_cand = _scatter_add(candidate_rep, graph_rep, idx_flat)
    return (out_cand, graph_rep)


# speedup vs baseline: 1.5541x; 1.5541x over previous
import functools

import jax
import jax.numpy as jnp
from jax import lax
from jax.experimental import pallas as pl
from jax.experimental.pallas import tpu as pltpu
from jax.experimental.pallas import tpu_sc as plsc

_NC = 2
_NS = 16
_W = 16
_G = 312
_ZH = _G // 2


def _make_plan(b):
    rps = ((b // _NS) // 8) * 8
    tail = b - _NS * rps
    nch = rps // _G
    rem = rps - nch * _G
    assert tail % 8 == 0 and tail <= 128 and rem <= 128
    assert nch >= 2 and nch % 2 == 0
    return rps, tail, nch, rem


def _sc_body(plan, cand, graph, idx, iota, out, acc, a0, a1, b0, b1, i0, i1,
             d0, d1, idx_rem, idx_tail, idn_rem, idn_tail, zsrc,
             la0, la1, sc0, sc1, zs, wb0, wb1):
    rps, tail, nch, rem = plan
    npass = cand.shape[1] // (_NC * _W)
    c = lax.axis_index("c")
    s = lax.axis_index("s")
    row0 = s * rps
    t0 = _NS * rps
    a_ = (a0, a1)
    b_ = (b0, b1)
    i_ = (i0, i1)
    d_ = (d0, d1)
    la = (la0, la1)
    sc = (sc0, sc1)
    wb = (wb0, wb1)

    if rem:
        pltpu.sync_copy(idx.at[pl.ds(row0 + nch * _G, rem)], idx_rem)
        pltpu.sync_copy(iota.at[pl.ds(row0 + nch * _G, rem)], idn_rem)
    if tail:
        @pl.when(s == _NS - 1)
        def _():
            pltpu.sync_copy(idx.at[pl.ds(t0, tail)], idx_tail)
            pltpu.sync_copy(iota.at[pl.ds(t0, tail)], idn_tail)

    @plsc.parallel_loop(0, _ZH, step=8, unroll=2)
    def _zero(i):
        for k in range(8):
            zsrc[i + k, :] = jnp.zeros((_W,), jnp.float32)

    def _add_rows(dst, src, n):
        @plsc.parallel_loop(0, n, step=8, unroll=2)
        def _body(i):
            for k in range(8):
                dst[i + k, :] = dst[i + k, :] + src[i + k, :]

    def one_pass(p, carry):
        col0 = (c * npass + p) * _W

        @pl.when(p == 0)
        def _():
            def zfill(k, carry2):
                for h in (0, 1):
                    pltpu.async_copy(
                        zsrc, acc.at[pl.ds(row0 + k * _G + h * _ZH, _ZH), :],
                        zs)
                return carry2
            lax.fori_loop(0, nch, zfill, 0)
            if rem:
                pltpu.async_copy(zsrc.at[pl.ds(0, rem), :],
                                 acc.at[pl.ds(row0 + nch * _G, rem), :], zs)
            if tail:
                @pl.when(s == _NS - 1)
                def _():
                    pltpu.async_copy(zsrc.at[pl.ds(0, tail), :],
                                     acc.at[pl.ds(t0, tail), :], zs)

        @pl.when(p > 0)
        def _():
            colq = (c * npass + p - 1) * _W
            for u in (0, 1):
                pltpu.make_async_copy(
                    b_[u], out.at[pl.ds(row0, _G), pl.ds(colq, _W)],
                    wb[u]).wait()

        def zdrain(k, carry2):
            for h in (0, 1):
                pltpu.make_async_copy(
                    zsrc, acc.at[pl.ds(row0 + k * _G + h * _ZH, _ZH), :],
                    zs).wait()
            return carry2
        lax.fori_loop(0, nch, zdrain, 0)
        if rem:
            pltpu.make_async_copy(zsrc.at[pl.ds(0, rem), :],
                                  acc.at[pl.ds(row0 + nch * _G, rem), :],
                                  zs).wait()
        if tail:
            @pl.when(s == _NS - 1)
            def _():
                pltpu.make_async_copy(zsrc.at[pl.ds(0, tail), :],
                                      acc.at[pl.ds(t0, tail), :], zs).wait()

        def issue_loads(j, u):
            base = row0 + j * _G
            pltpu.async_copy(idx.at[pl.ds(base, _G)], i_[u], la[u])
            pltpu.async_copy(iota.at[pl.ds(base, _G)], d_[u], la[u])
            pltpu.async_copy(cand.at[pl.ds(base, _G), pl.ds(col0, _W)],
                             a_[u], la[u])
            pltpu.async_copy(graph.at[pl.ds(base, _G), pl.ds(col0, _W)],
                             b_[u], la[u])

        def wait_loads(j, u):
            base = row0 + j * _G
            pltpu.make_async_copy(idx.at[pl.ds(base, _G)], i_[u], la[u]).wait()
            pltpu.make_async_copy(iota.at[pl.ds(base, _G)], d_[u],
                                  la[u]).wait()
            pltpu.make_async_copy(cand.at[pl.ds(base, _G), pl.ds(col0, _W)],
                                  a_[u], la[u]).wait()
            pltpu.make_async_copy(graph.at[pl.ds(base, _G), pl.ds(col0, _W)],
                                  b_[u], la[u]).wait()

        def issue_scatter(u):
            pltpu.async_copy(b_[u], acc.at[i_[u]], sc[u], add=True)
            pltpu.async_copy(a_[u], acc.at[i_[u]], sc[u], add=True)
            pltpu.async_copy(a_[u], acc.at[d_[u]], sc[u], add=True)

        def wait_scatter(u):
            pltpu.make_async_copy(b_[u], acc.at[i_[u]], sc[u]).wait()
            pltpu.make_async_copy(a_[u], acc.at[i_[u]], sc[u]).wait()
            pltpu.make_async_copy(a_[u], acc.at[d_[u]], sc[u]).wait()

        issue_loads(0, 0)
        plsc.subcore_barrier()
        wait_loads(0, 0)
        issue_loads(1, 1)
        issue_scatter(0)

        def pair(j2, carry2):
            j = 1 + 2 * j2
            wait_loads(j, 1)
            wait_scatter(0)
            issue_loads(j + 1, 0)
            issue_scatter(1)
            wait_loads(j + 1, 0)
            wait_scatter(1)
            issue_loads(j + 2, 1)
            issue_scatter(0)
            return carry2
        lax.fori_loop(0, (nch - 2) // 2, pair, 0)

        wait_loads(nch - 1, 1)
        wait_scatter(0)
        issue_scatter(1)

        if rem:
            roff = row0 + nch * _G
            pltpu.sync_copy(cand.at[pl.ds(roff, rem), pl.ds(col0, _W)],
                            a0.at[pl.ds(0, rem), :])
            pltpu.sync_copy(graph.at[pl.ds(roff, rem), pl.ds(col0, _W)],
                            b0.at[pl.ds(0, rem), :])
            pltpu.sync_copy(b0.at[pl.ds(0, rem), :],
                            acc.at[idx_rem], add=True)
            pltpu.sync_copy(a0.at[pl.ds(0, rem), :],
                            acc.at[idx_rem], add=True)
            pltpu.sync_copy(a0.at[pl.ds(0, rem), :],
                            acc.at[idn_rem], add=True)

        if tail:
            @pl.when(s == _NS - 1)
            def _():
                pltpu.sync_copy(cand.at[pl.ds(t0, tail), pl.ds(col0, _W)],
                                a0.at[pl.ds(0, tail), :])
                pltpu.sync_copy(graph.at[pl.ds(t0, tail), pl.ds(col0, _W)],
                                b0.at[pl.ds(0, tail), :])
                pltpu.sync_copy(b0.at[pl.ds(0, tail), :],
                                acc.at[idx_tail], add=True)
                pltpu.sync_copy(a0.at[pl.ds(0, tail), :],
                                acc.at[idx_tail], add=True)
                pltpu.sync_copy(a0.at[pl.ds(0, tail), :],
                                acc.at[idn_tail], add=True)

        wait_scatter(1)
        plsc.subcore_barrier()

        if rem:
            roff = row0 + nch * _G
            pltpu.sync_copy(acc.at[pl.ds(roff, rem), :],
                            b1.at[pl.ds(0, rem), :])
            pltpu.sync_copy(b1.at[pl.ds(0, rem), :],
                            out.at[pl.ds(roff, rem), pl.ds(col0, _W)])
            pltpu.async_copy(zsrc.at[pl.ds(0, rem), :],
                             acc.at[pl.ds(roff, rem), :], zs)
        if tail:
            @pl.when(s == _NS - 1)
            def _():
                pltpu.sync_copy(acc.at[pl.ds(t0, tail), :],
                                b1.at[pl.ds(0, tail), :])
                pltpu.sync_copy(b1.at[pl.ds(0, tail), :],
                                out.at[pl.ds(t0, tail), pl.ds(col0, _W)])
                pltpu.async_copy(zsrc.at[pl.ds(0, tail), :],
                                 acc.at[pl.ds(t0, tail), :], zs)

        def wb_store(k, u):
            pltpu.async_copy(b_[u], out.at[pl.ds(row0 + k * _G, _G),
                                           pl.ds(col0, _W)], wb[u])

        def wb_wait_store(k, u):
            pltpu.make_async_copy(b_[u], out.at[pl.ds(row0 + k * _G, _G),
                                                pl.ds(col0, _W)],
                                  wb[u]).wait()

        def wb_step(k, u):
            pltpu.sync_copy(acc.at[pl.ds(row0 + k * _G, _G), :], b_[u])
            wb_store(k, u)
            for h in (0, 1):
                pltpu.async_copy(
                    zsrc, acc.at[pl.ds(row0 + k * _G + h * _ZH, _ZH), :], zs)

        wb_step(0, 0)
        wb_step(1, 1)

        def wb_pair(k2, carry2):
            k = 2 + 2 * k2
            wb_wait_store(k - 2, 0)
            wb_step(k, 0)
            wb_wait_store(k - 1, 1)
            wb_step(k + 1, 1)
            return carry2
        lax.fori_loop(0, (nch - 2) // 2, wb_pair, 0)
        return carry

    lax.fori_loop(0, npass, one_pass, 0)

    colz = (c * npass + npass - 1) * _W
    for u in (0, 1):
        pltpu.make_async_copy(b_[u], out.at[pl.ds(row0, _G), pl.ds(colz, _W)],
                              wb[u]).wait()

    def zdrain_f(k, carry2):
        for h in (0, 1):
            pltpu.make_async_copy(
                zsrc, acc.at[pl.ds(row0 + k * _G + h * _ZH, _ZH), :],
                zs).wait()
        return carry2
    lax.fori_loop(0, nch, zdrain_f, 0)
    if rem:
        pltpu.make_async_copy(zsrc.at[pl.ds(0, rem), :],
                              acc.at[pl.ds(row0 + nch * _G, rem), :],
                              zs).wait()
    if tail:
        @pl.when(s == _NS - 1)
        def _():
            pltpu.make_async_copy(zsrc.at[pl.ds(0, tail), :],
                                  acc.at[pl.ds(t0, tail), :], zs).wait()


@jax.jit
def _scatter_add(cand, graph, idx_flat):
    b, d = cand.shape
    plan = _make_plan(b)
    rps, tail, nch, rem = plan
    iota = jnp.arange(b, dtype=jnp.int32)
    mesh = plsc.VectorSubcoreMesh(core_axis_name="c", subcore_axis_name="s")
    f = pl.kernel(
        functools.partial(_sc_body, plan),
        out_type=jax.ShapeDtypeStruct((b, d), jnp.float32),
        mesh=mesh,
        compiler_params=pltpu.CompilerParams(use_tc_tiling_on_sc=False),
        scratch_types=[
            pltpu.VMEM_SHARED((b, _W), jnp.float32),
            pltpu.VMEM((_G, _W), jnp.float32),
            pltpu.VMEM((_G, _W), jnp.float32),
            pltpu.VMEM((_G, _W), jnp.float32),
            pltpu.VMEM((_G, _W), jnp.float32),
            pltpu.VMEM((_G,), jnp.int32),
            pltpu.VMEM((_G,), jnp.int32),
            pltpu.VMEM((_G,), jnp.int32),
            pltpu.VMEM((_G,), jnp.int32),
            pltpu.VMEM((max(rem, 1),), jnp.int32),
            pltpu.VMEM((max(tail, 1),), jnp.int32),
            pltpu.VMEM((max(rem, 1),), jnp.int32),
            pltpu.VMEM((max(tail, 1),), jnp.int32),
            pltpu.VMEM((_ZH, _W), jnp.float32),
            pltpu.SemaphoreType.DMA,
            pltpu.SemaphoreType.DMA,
            pltpu.SemaphoreType.DMA,
            pltpu.SemaphoreType.DMA,
            pltpu.SemaphoreType.DMA,
            pltpu.SemaphoreType.DMA,
            pltpu.SemaphoreType.DMA,
        ],
    )
    return f(cand, graph, idx_flat, iota)


def kernel(candidate_rep, graph_rep, graph_sizes, put_indices):
    del graph_sizes
    idx_flat = put_indices.reshape(-1)
    out_cand = _scatter_add(candidate_rep, graph_rep, idx_flat)
    return (out_cand, graph_rep)


# speedup vs baseline: 1.5557x; 1.0010x over previous
import functools

import jax
import jax.numpy as jnp
from jax import lax
from jax.experimental import pallas as pl
from jax.experimental.pallas import tpu as pltpu
from jax.experimental.pallas import tpu_sc as plsc

_NC = 2
_NS = 16
_W = 16
_G = 312
_ZH = _G // 2


def _make_plan(b):
    rps = ((b // _NS) // 8) * 8
    tail = b - _NS * rps
    nch = rps // _G
    rem = rps - nch * _G
    assert tail % 8 == 0 and tail <= 128 and rem <= 128
    assert nch >= 2 and nch % 2 == 0
    return rps, tail, nch, rem


def _sc_body(plan, cand, graph, idx, iota, out, acc, a0, a1, b0, b1, i0, i1,
             d0, d1, idx_rem, idx_tail, idn_rem, idn_tail, zsrc,
             la0, la1, sc0, sc1, zs, wb0, wb1):
    rps, tail, nch, rem = plan
    npass = cand.shape[1] // (_NC * _W)
    c = lax.axis_index("c")
    s = lax.axis_index("s")
    row0 = s * rps
    t0 = _NS * rps
    a_ = (a0, a1)
    b_ = (b0, b1)
    i_ = (i0, i1)
    d_ = (d0, d1)
    la = (la0, la1)
    sc = (sc0, sc1)
    wb = (wb0, wb1)

    if rem:
        pltpu.sync_copy(idx.at[pl.ds(row0 + nch * _G, rem)], idx_rem)
        pltpu.sync_copy(iota.at[pl.ds(row0 + nch * _G, rem)], idn_rem)
    if tail:
        @pl.when(s == _NS - 1)
        def _():
            pltpu.sync_copy(idx.at[pl.ds(t0, tail)], idx_tail)
            pltpu.sync_copy(iota.at[pl.ds(t0, tail)], idn_tail)

    @plsc.parallel_loop(0, _ZH, step=8, unroll=2)
    def _zero(i):
        for k in range(8):
            zsrc[i + k, :] = jnp.zeros((_W,), jnp.float32)

    def one_pass(p, carry):
        col0 = (c * npass + p) * _W

        @pl.when(p == 0)
        def _():
            def zfill(k, carry2):
                for h in (0, 1):
                    pltpu.async_copy(
                        zsrc, acc.at[pl.ds(row0 + k * _G + h * _ZH, _ZH), :],
                        zs)
                return carry2
            lax.fori_loop(0, nch, zfill, 0)
            if rem:
                pltpu.async_copy(zsrc.at[pl.ds(0, rem), :],
                                 acc.at[pl.ds(row0 + nch * _G, rem), :], zs)
            if tail:
                @pl.when(s == _NS - 1)
                def _():
                    pltpu.async_copy(zsrc.at[pl.ds(0, tail), :],
                                     acc.at[pl.ds(t0, tail), :], zs)

        @pl.when(p > 0)
        def _():
            colq = (c * npass + p - 1) * _W
            for u in (0, 1):
                pltpu.make_async_copy(
                    b_[u], out.at[pl.ds(row0, _G), pl.ds(colq, _W)],
                    wb[u]).wait()

        def zdrain(k, carry2):
            for h in (0, 1):
                pltpu.make_async_copy(
                    zsrc, acc.at[pl.ds(row0 + k * _G + h * _ZH, _ZH), :],
                    zs).wait()
            return carry2
        lax.fori_loop(0, nch, zdrain, 0)
        if rem:
            pltpu.make_async_copy(zsrc.at[pl.ds(0, rem), :],
                                  acc.at[pl.ds(row0 + nch * _G, rem), :],
                                  zs).wait()
        if tail:
            @pl.when(s == _NS - 1)
            def _():
                pltpu.make_async_copy(zsrc.at[pl.ds(0, tail), :],
                                      acc.at[pl.ds(t0, tail), :], zs).wait()

        def issue_loads(j, u):
            base = row0 + j * _G
            pltpu.async_copy(idx.at[pl.ds(base, _G)], i_[u], la[u])
            pltpu.async_copy(iota.at[pl.ds(base, _G)], d_[u], la[u])
            pltpu.async_copy(cand.at[pl.ds(base, _G), pl.ds(col0, _W)],
                             a_[u], la[u])
            pltpu.async_copy(graph.at[pl.ds(base, _G), pl.ds(col0, _W)],
                             b_[u], la[u])

        def wait_loads(j, u):
            base = row0 + j * _G
            pltpu.make_async_copy(idx.at[pl.ds(base, _G)], i_[u], la[u]).wait()
            pltpu.make_async_copy(iota.at[pl.ds(base, _G)], d_[u],
                                  la[u]).wait()
            pltpu.make_async_copy(cand.at[pl.ds(base, _G), pl.ds(col0, _W)],
                                  a_[u], la[u]).wait()
            pltpu.make_async_copy(graph.at[pl.ds(base, _G), pl.ds(col0, _W)],
                                  b_[u], la[u]).wait()

        def issue_scatter(u):
            pltpu.async_copy(b_[u], acc.at[i_[u]], sc[u], add=True)
            pltpu.async_copy(a_[u], acc.at[i_[u]], sc[u], add=True)
            pltpu.async_copy(a_[u], acc.at[d_[u]], sc[u], add=True)

        def wait_scatter(u):
            pltpu.make_async_copy(b_[u], acc.at[i_[u]], sc[u]).wait()
            pltpu.make_async_copy(a_[u], acc.at[i_[u]], sc[u]).wait()
            pltpu.make_async_copy(a_[u], acc.at[d_[u]], sc[u]).wait()

        issue_loads(0, 0)
        plsc.subcore_barrier()
        wait_loads(0, 0)
        issue_loads(1, 1)
        issue_scatter(0)

        def pair(j2, carry2):
            j = 1 + 2 * j2
            wait_loads(j, 1)
            wait_scatter(0)
            issue_loads(j + 1, 0)
            issue_scatter(1)
            wait_loads(j + 1, 0)
            wait_scatter(1)
            issue_loads(j + 2, 1)
            issue_scatter(0)
            return carry2
        lax.fori_loop(0, (nch - 2) // 2, pair, 0)

        wait_loads(nch - 1, 1)
        wait_scatter(0)
        issue_scatter(1)

        if rem:
            roff = row0 + nch * _G
            pltpu.sync_copy(cand.at[pl.ds(roff, rem), pl.ds(col0, _W)],
                            a0.at[pl.ds(0, rem), :])
            pltpu.sync_copy(graph.at[pl.ds(roff, rem), pl.ds(col0, _W)],
                            b0.at[pl.ds(0, rem), :])
            pltpu.sync_copy(b0.at[pl.ds(0, rem), :],
                            acc.at[idx_rem], add=True)
            pltpu.sync_copy(a0.at[pl.ds(0, rem), :],
                            acc.at[idx_rem], add=True)
            pltpu.sync_copy(a0.at[pl.ds(0, rem), :],
                            acc.at[idn_rem], add=True)

        if tail:
            @pl.when(s == _NS - 1)
            def _():
                pltpu.sync_copy(cand.at[pl.ds(t0, tail), pl.ds(col0, _W)],
                                a0.at[pl.ds(0, tail), :])
                pltpu.sync_copy(graph.at[pl.ds(t0, tail), pl.ds(col0, _W)],
                                b0.at[pl.ds(0, tail), :])
                pltpu.sync_copy(b0.at[pl.ds(0, tail), :],
                                acc.at[idx_tail], add=True)
                pltpu.sync_copy(a0.at[pl.ds(0, tail), :],
                                acc.at[idx_tail], add=True)
                pltpu.sync_copy(a0.at[pl.ds(0, tail), :],
                                acc.at[idn_tail], add=True)

        wait_scatter(1)
        plsc.subcore_barrier()

        if rem:
            roff = row0 + nch * _G
            pltpu.sync_copy(acc.at[pl.ds(roff, rem), :],
                            b1.at[pl.ds(0, rem), :])
            pltpu.sync_copy(b1.at[pl.ds(0, rem), :],
                            out.at[pl.ds(roff, rem), pl.ds(col0, _W)])
            pltpu.async_copy(zsrc.at[pl.ds(0, rem), :],
                             acc.at[pl.ds(roff, rem), :], zs)
        if tail:
            @pl.when(s == _NS - 1)
            def _():
                pltpu.sync_copy(acc.at[pl.ds(t0, tail), :],
                                b1.at[pl.ds(0, tail), :])
                pltpu.sync_copy(b1.at[pl.ds(0, tail), :],
                                out.at[pl.ds(t0, tail), pl.ds(col0, _W)])
                pltpu.async_copy(zsrc.at[pl.ds(0, tail), :],
                                 acc.at[pl.ds(t0, tail), :], zs)

        def wb_store(k, u):
            pltpu.async_copy(b_[u], out.at[pl.ds(row0 + k * _G, _G),
                                           pl.ds(col0, _W)], wb[u])

        def wb_wait_store(k, u):
            pltpu.make_async_copy(b_[u], out.at[pl.ds(row0 + k * _G, _G),
                                                pl.ds(col0, _W)],
                                  wb[u]).wait()

        def wb_step(k, u):
            pltpu.sync_copy(acc.at[pl.ds(row0 + k * _G, _G), :], b_[u])
            wb_store(k, u)
            for h in (0, 1):
                pltpu.async_copy(
                    zsrc, acc.at[pl.ds(row0 + k * _G + h * _ZH, _ZH), :], zs)

        wb_step(0, 0)
        wb_step(1, 1)

        def wb_pair(k2, carry2):
            k = 2 + 2 * k2
            wb_wait_store(k - 2, 0)
            wb_step(k, 0)
            wb_wait_store(k - 1, 1)
            wb_step(k + 1, 1)
            return carry2
        lax.fori_loop(0, (nch - 2) // 2, wb_pair, 0)
        return carry

    lax.fori_loop(0, npass, one_pass, 0)

    colz = (c * npass + npass - 1) * _W
    for u in (0, 1):
        pltpu.make_async_copy(b_[u], out.at[pl.ds(row0, _G), pl.ds(colz, _W)],
                              wb[u]).wait()

    def zdrain_f(k, carry2):
        for h in (0, 1):
            pltpu.make_async_copy(
                zsrc, acc.at[pl.ds(row0 + k * _G + h * _ZH, _ZH), :],
                zs).wait()
        return carry2
    lax.fori_loop(0, nch, zdrain_f, 0)
    if rem:
        pltpu.make_async_copy(zsrc.at[pl.ds(0, rem), :],
                              acc.at[pl.ds(row0 + nch * _G, rem), :],
                              zs).wait()
    if tail:
        @pl.when(s == _NS - 1)
        def _():
            pltpu.make_async_copy(zsrc.at[pl.ds(0, tail), :],
                                  acc.at[pl.ds(t0, tail), :], zs).wait()


@jax.jit
def _scatter_add(cand, graph, idx_flat):
    b, d = cand.shape
    plan = _make_plan(b)
    rps, tail, nch, rem = plan
    iota = jnp.arange(b, dtype=jnp.int32)
    mesh = plsc.VectorSubcoreMesh(core_axis_name="c", subcore_axis_name="s")
    f = pl.kernel(
        functools.partial(_sc_body, plan),
        out_type=jax.ShapeDtypeStruct((b, d), jnp.float32),
        mesh=mesh,
        compiler_params=pltpu.CompilerParams(use_tc_tiling_on_sc=False),
        scratch_types=[
            pltpu.VMEM_SHARED((b, _W), jnp.float32),
            pltpu.VMEM((_G, _W), jnp.float32),
            pltpu.VMEM((_G, _W), jnp.float32),
            pltpu.VMEM((_G, _W), jnp.float32),
            pltpu.VMEM((_G, _W), jnp.float32),
            pltpu.VMEM((_G,), jnp.int32),
            pltpu.VMEM((_G,), jnp.int32),
            pltpu.VMEM((_G,), jnp.int32),
            pltpu.VMEM((_G,), jnp.int32),
            pltpu.VMEM((max(rem, 1),), jnp.int32),
            pltpu.VMEM((max(tail, 1),), jnp.int32),
            pltpu.VMEM((max(rem, 1),), jnp.int32),
            pltpu.VMEM((max(tail, 1),), jnp.int32),
            pltpu.VMEM((_ZH, _W), jnp.float32),
            pltpu.SemaphoreType.DMA,
            pltpu.SemaphoreType.DMA,
            pltpu.SemaphoreType.DMA,
            pltpu.SemaphoreType.DMA,
            pltpu.SemaphoreType.DMA,
            pltpu.SemaphoreType.DMA,
            pltpu.SemaphoreType.DMA,
        ],
    )
    return f(cand, graph, idx_flat, iota)


def kernel(candidate_rep, graph_rep, graph_sizes, put_indices):
    del graph_sizes
    idx_flat = put_indices.reshape(-1)
    out_cand = _scatter_add(candidate_rep, graph_rep, idx_flat)
    return (out_cand, graph_rep)
